# baseline (speedup 1.0000x reference)
import jax
import jax.numpy as jnp
from jax import lax
from jax.experimental import pallas as pl
from jax.experimental.pallas import tpu as pltpu
from jax.experimental.pallas import tpu_sc as plsc

B = 16384
EMB = 32
MLP = 128

_info = plsc.get_sparse_core_info()
_NC, _NS = _info.num_cores, _info.num_subcores
_NW = _NC * _NS
_RPW = B // _NW
_CH = 128
_NCH = _RPW // _CH


def _sc_body(user_h, item_h, ug_h, ig_h, um_h, im_h,
             oug_h, oig_h, oum_h, oim_h,
             idx_u, idx_i, buf_m,
             sem_m, sem_gu, sem_gi):
    c = lax.axis_index("c")
    s = lax.axis_index("s")
    wid = s * _NC + c
    base = wid * _RPW
    pltpu.sync_copy(user_h.at[pl.ds(wid * _NCH, _NCH)], idx_u)
    pltpu.sync_copy(item_h.at[pl.ds(wid * _NCH, _NCH)], idx_i)
    um_cps = [pltpu.async_copy(um_h.at[idx_u.at[j]],
                               buf_m.at[pl.ds(j * _CH, _CH)], sem_m)
              for j in range(_NCH)]

    def fire_rows(tab_h, idx, out_h, sem):
        for j in range(_NCH):
            def body(g, _):
                vec = idx[j, pl.ds(g * 16, 16)]
                for l in range(16):
                    row = vec[l]
                    pltpu.async_copy(
                        tab_h.at[pl.ds(row, 1)],
                        out_h.at[pl.ds(base + j * _CH + g * 16 + l, 1)], sem)
                return 0
            lax.fori_loop(0, _CH // 16, body, 0)

    fire_rows(ug_h, idx_u, oug_h, sem_gu)
    fire_rows(ig_h, idx_i, oig_h, sem_gi)

    for cp in um_cps:
        cp.wait()
    pltpu.sync_copy(buf_m, oum_h.at[pl.ds(base, _RPW)])
    im_cps = [pltpu.async_copy(im_h.at[idx_i.at[j]],
                               buf_m.at[pl.ds(j * _CH, _CH)], sem_m)
              for j in range(_NCH)]

    pltpu.make_async_copy(ug_h.at[pl.ds(0, _RPW)],
                          oug_h.at[pl.ds(base, _RPW)], sem_gu).wait()
    pltpu.make_async_copy(ig_h.at[pl.ds(0, _RPW)],
                          oig_h.at[pl.ds(base, _RPW)], sem_gi).wait()

    for cp in im_cps:
        cp.wait()
    pltpu.sync_copy(buf_m, oim_h.at[pl.ds(base, _RPW)])


def _sc_gather(user2d, item2d, ug, ig, um, im):
    mesh = plsc.VectorSubcoreMesh(core_axis_name="c", subcore_axis_name="s")
    f32 = jnp.float32
    out_type = [
        jax.ShapeDtypeStruct((B, EMB), f32),
        jax.ShapeDtypeStruct((B, EMB), f32),
        jax.ShapeDtypeStruct((B, MLP), f32),
        jax.ShapeDtypeStruct((B, MLP), f32),
    ]
    scratch = [
        pltpu.VMEM((_NCH, _CH), jnp.int32),
        pltpu.VMEM((_NCH, _CH), jnp.int32),
        pltpu.VMEM((_RPW, MLP), f32),
        pltpu.SemaphoreType.DMA,
        pltpu.SemaphoreType.DMA,
        pltpu.SemaphoreType.DMA,
    ]
    return pl.kernel(
        _sc_body, mesh=mesh, out_type=out_type, scratch_types=scratch,
    )(user2d, item2d, ug, ig, um, im)


_TB = 1024


def _tc_body(ueg, ieg, uem, iem, w1a, w1b, b1, w2, b2, w3, b3,
             wpg, wph, bp, out):
    f32 = jnp.float32
    g = ueg[...] * ieg[...]
    h = jnp.maximum(
        jnp.dot(uem[...], w1a[...], preferred_element_type=f32)
        + jnp.dot(iem[...], w1b[...], preferred_element_type=f32)
        + b1[...], 0.0)
    h = jnp.maximum(jnp.dot(h, w2[...], preferred_element_type=f32)
                    + b2[...], 0.0)
    h = jnp.maximum(jnp.dot(h, w3[...], preferred_element_type=f32)
                    + b3[...], 0.0)
    pred = (jnp.dot(g, wpg[...], preferred_element_type=f32)
            + jnp.dot(h, wph[...], preferred_element_type=f32)
            + bp[...])
    out[...] = pred


def _tc_mlp(ueg, ieg, uem, iem, w1a, w1b, b1, w2, b2, w3, b3, wpg, wph, bp2):
    def rows(d):
        return pl.BlockSpec((_TB, d), lambda i: (i, 0))

    def full2(a, b):
        return pl.BlockSpec((a, b), lambda i: (0, 0))

    def full1(a):
        return pl.BlockSpec((a,), lambda i: (0,))

    return pl.pallas_call(
        _tc_body,
        grid=(B // _TB,),
        in_specs=[
            rows(EMB), rows(EMB), rows(MLP), rows(MLP),
            full2(MLP, 128), full2(MLP, 128), full1(128),
            full2(128, 64), full1(64),
            full2(64, 32), full1(32),
            full2(EMB, 1), full2(32, 1), full2(1, 1),
        ],
        out_specs=pl.BlockSpec((_TB, 1), lambda i: (i, 0)),
        out_shape=jax.ShapeDtypeStruct((B, 1), jnp.float32),
    )(ueg, ieg, uem, iem, w1a, w1b, b1, w2, b2, w3, b3, wpg, wph, bp2)


def kernel(user, item, user_gmf, item_gmf, user_mlp, item_mlp,
           W1, b1, W2, b2, W3, b3, Wp, bp):
    user2d = user.astype(jnp.int32).reshape(_NW * _NCH, _CH)
    item2d = item.astype(jnp.int32).reshape(_NW * _NCH, _CH)
    ueg, ieg, uem, iem = _sc_gather(user2d, item2d, user_gmf, item_gmf,
                                    user_mlp, item_mlp)
    W1a, W1b = W1[:MLP], W1[MLP:]
    Wpg, Wph = Wp[:EMB], Wp[EMB:]
    out = _tc_mlp(ueg, ieg, uem, iem, W1a, W1b, b1, W2, b2, W3, b3,
                  Wpg, Wph, bp.reshape(1, 1))
    return out.reshape(B)


# speedup vs baseline: 1.7504x; 1.7504x over previous
import jax
import jax.numpy as jnp
from jax import lax
from jax.experimental import pallas as pl
from jax.experimental.pallas import tpu as pltpu
from jax.experimental.pallas import tpu_sc as plsc

B = 16384
EMB = 32
MLP = 128

_info = plsc.get_sparse_core_info()
_NC, _NS = _info.num_cores, _info.num_subcores
_NW = _NC * _NS
_RPW = B // _NW
_CH = 128
_NCH = _RPW // _CH


def _sc_body(user_h, item_h, ug_h, ig_h, um_h, im_h,
             oug_h, oig_h, oum_h, oim_h,
             idx_u, idx_i, buf_m, buf_g,
             sem_m, sem_g):
    c = lax.axis_index("c")
    s = lax.axis_index("s")
    wid = s * _NC + c
    base = wid * _RPW
    pltpu.sync_copy(user_h.at[pl.ds(wid * _NCH, _NCH)], idx_u)
    pltpu.sync_copy(item_h.at[pl.ds(wid * _NCH, _NCH)], idx_i)
    um_cps = [pltpu.async_copy(um_h.at[idx_u.at[j]],
                               buf_m.at[pl.ds(j * _CH, _CH)], sem_m)
              for j in range(_NCH)]

    def gather_gmf(tab_h, idx, out_h):
        for j in range(_NCH):
            def body(g, _):
                vec = idx[j, pl.ds(g * 16, 16)]
                for l in range(16):
                    pltpu.async_copy(tab_h.at[pl.ds(vec[l], 1)],
                                     buf_g.at[pl.ds(g * 16 + l, 1)], sem_g)
                return 0
            lax.fori_loop(0, _CH // 16, body, 0)
            pltpu.make_async_copy(tab_h.at[pl.ds(0, _CH)], buf_g, sem_g).wait()
            pltpu.sync_copy(buf_g, out_h.at[pl.ds(base + j * _CH, _CH)])

    gather_gmf(ug_h, idx_u, oug_h)
    gather_gmf(ig_h, idx_i, oig_h)

    for cp in um_cps:
        cp.wait()
    pltpu.sync_copy(buf_m, oum_h.at[pl.ds(base, _RPW)])
    im_cps = [pltpu.async_copy(im_h.at[idx_i.at[j]],
                               buf_m.at[pl.ds(j * _CH, _CH)], sem_m)
              for j in range(_NCH)]
    for cp in im_cps:
        cp.wait()
    pltpu.sync_copy(buf_m, oim_h.at[pl.ds(base, _RPW)])


def _sc_gather(user2d, item2d, ug, ig, um, im):
    mesh = plsc.VectorSubcoreMesh(core_axis_name="c", subcore_axis_name="s")
    f32 = jnp.float32
    out_type = [
        jax.ShapeDtypeStruct((B, EMB), f32),
        jax.ShapeDtypeStruct((B, EMB), f32),
        jax.ShapeDtypeStruct((B, MLP), f32),
        jax.ShapeDtypeStruct((B, MLP), f32),
    ]
    scratch = [
        pltpu.VMEM((_NCH, _CH), jnp.int32),
        pltpu.VMEM((_NCH, _CH), jnp.int32),
        pltpu.VMEM((_RPW, MLP), f32),
        pltpu.VMEM((_CH, EMB), f32),
        pltpu.SemaphoreType.DMA,
        pltpu.SemaphoreType.DMA,
    ]
    return pl.kernel(
        _sc_body, mesh=mesh, out_type=out_type, scratch_types=scratch,
    )(user2d, item2d, ug, ig, um, im)


_TB = 1024


def _tc_body(ueg, ieg, uem, iem, w1a, w1b, b1, w2, b2, w3, b3,
             wpg, wph, bp, out):
    f32 = jnp.float32
    g = ueg[...] * ieg[...]
    h = jnp.maximum(
        jnp.dot(uem[...], w1a[...], preferred_element_type=f32)
        + jnp.dot(iem[...], w1b[...], preferred_element_type=f32)
        + b1[...], 0.0)
    h = jnp.maximum(jnp.dot(h, w2[...], preferred_element_type=f32)
                    + b2[...], 0.0)
    h = jnp.maximum(jnp.dot(h, w3[...], preferred_element_type=f32)
                    + b3[...], 0.0)
    pred = (jnp.dot(g, wpg[...], preferred_element_type=f32)
            + jnp.dot(h, wph[...], preferred_element_type=f32)
            + bp[...])
    out[...] = pred


def _tc_mlp(ueg, ieg, uem, iem, w1a, w1b, b1, w2, b2, w3, b3, wpg, wph, bp2):
    def rows(d):
        return pl.BlockSpec((_TB, d), lambda i: (i, 0))

    def full2(a, b):
        return pl.BlockSpec((a, b), lambda i: (0, 0))

    def full1(a):
        return pl.BlockSpec((a,), lambda i: (0,))

    return pl.pallas_call(
        _tc_body,
        grid=(B // _TB,),
        in_specs=[
            rows(EMB), rows(EMB), rows(MLP), rows(MLP),
            full2(MLP, 128), full2(MLP, 128), full1(128),
            full2(128, 64), full1(64),
            full2(64, 32), full1(32),
            full2(EMB, 1), full2(32, 1), full2(1, 1),
        ],
        out_specs=pl.BlockSpec((_TB, 1), lambda i: (i, 0)),
        out_shape=jax.ShapeDtypeStruct((B, 1), jnp.float32),
    )(ueg, ieg, uem, iem, w1a, w1b, b1, w2, b2, w3, b3, wpg, wph, bp2)


def kernel(user, item, user_gmf, item_gmf, user_mlp, item_mlp,
           W1, b1, W2, b2, W3, b3, Wp, bp):
    user2d = user.astype(jnp.int32).reshape(_NW * _NCH, _CH)
    item2d = item.astype(jnp.int32).reshape(_NW * _NCH, _CH)
    ueg, ieg, uem, iem = _sc_gather(user2d, item2d, user_gmf, item_gmf,
                                    user_mlp, item_mlp)
    W1a, W1b = W1[:MLP], W1[MLP:]
    Wpg, Wph = Wp[:EMB], Wp[EMB:]
    out = _tc_mlp(ueg, ieg, uem, iem, W1a, W1b, b1, W2, b2, W3, b3,
                  Wpg, Wph, bp.reshape(1, 1))
    return out.reshape(B)
